# baseline (speedup 1.0000x reference)
import functools

import jax
import jax.numpy as jnp
from jax import lax
from jax.experimental import pallas as pl
from jax.experimental.pallas import tpu as pltpu
from jax.experimental.pallas import tpu_sc as plsc

E = 64
TOPK = 8
D = 2048
F = 1024
T = 4096
N = T * TOPK
TM = 256
NT = N // TM + E
NPAD = NT * TM
NW = 32
GCHUNK = 24
TCHUNK = 2


def _routing(top_k_index, top_k_weights):
    flat_e = top_k_index.reshape(-1).astype(jnp.int32)
    BB, LL = 256, N // 256
    e2 = flat_e.reshape(BB, LL).T
    onehot = (e2[:, :, None] ==
              jnp.arange(E, dtype=jnp.int32)[None, None, :]
              ).astype(jnp.bfloat16)
    tril_l = jnp.tril(jnp.ones((LL, LL), jnp.bfloat16), -1)
    cum_in_blk = jnp.dot(tril_l, onehot.reshape(LL, BB * E),
                         preferred_element_type=jnp.float32
                         ).reshape(LL, BB, E)
    bsum = jnp.sum(onehot.astype(jnp.float32), axis=0)
    tril_b = jnp.tril(jnp.ones((BB, BB), jnp.bfloat16), -1)
    bpre = jnp.dot(tril_b, bsum.astype(jnp.bfloat16),
                   preferred_element_type=jnp.float32)
    rank_full = cum_in_blk + bpre[None, :, :]
    rank = jnp.sum(rank_full * onehot.astype(jnp.float32),
                   axis=-1).T.reshape(N).astype(jnp.int32)
    counts = (bpre[BB - 1] + bsum[BB - 1]).astype(jnp.int32)
    tiles_per_e = (counts + TM - 1) // TM
    zero = jnp.zeros((1,), jnp.int32)
    pad_off = jnp.concatenate([zero, jnp.cumsum(tiles_per_e)[:-1] * TM])
    r = jnp.arange(N, dtype=jnp.int32)
    ppos = pad_off[flat_e] + rank
    spread = jnp.arange(NPAD, dtype=jnp.int32) % T
    tok_idx = spread.at[ppos].set(r // TOPK)
    wvec = jnp.zeros((NPAD,), jnp.float32).at[ppos].set(
        top_k_weights.reshape(-1))
    cidx = ppos
    cum_tiles = jnp.cumsum(tiles_per_e)
    total_tiles = cum_tiles[E - 1]
    te = jnp.searchsorted(cum_tiles, jnp.arange(NT, dtype=jnp.int32),
                          side="right").astype(jnp.int32)
    te = jnp.minimum(te, E - 1)
    last_e = te[jnp.maximum(total_tiles - 1, 0)]
    tile_expert = jnp.where(jnp.arange(NT, dtype=jnp.int32) < total_tiles,
                            te, last_e)
    tile_meta = jnp.concatenate([tile_expert, total_tiles[None]])
    return tok_idx, wvec, cidx, tile_meta


@functools.lru_cache(maxsize=None)
def _sc_gather_kernel():
    mesh = plsc.VectorSubcoreMesh(core_axis_name="c", subcore_axis_name="s")

    @functools.partial(
        pl.kernel,
        mesh=mesh,
        out_type=jax.ShapeDtypeStruct((NPAD, D), jnp.float32),
        scratch_types=[
            pltpu.VMEM((NPAD // NW,), jnp.int32),
            pltpu.VMEM((GCHUNK, D), jnp.float32),
            pltpu.VMEM((GCHUNK, D), jnp.float32),
            pltpu.SemaphoreType.DMA,
            pltpu.SemaphoreType.DMA,
        ],
    )
    def _sc_gather_rows(hidden_hbm, idx_hbm, out_hbm,
                        idx_all, rows0, rows1, sem0, sem1):
        wid = lax.axis_index("s") * 2 + lax.axis_index("c")
        rows_per_w = NPAD // NW
        nchunks = rows_per_w // GCHUNK
        base = wid * rows_per_w

        pltpu.sync_copy(idx_hbm.at[pl.ds(base, rows_per_w)], idx_all)

        def idx(i):
            return idx_all.at[pl.ds(i * GCHUNK, GCHUNK)]

        pltpu.async_copy(hidden_hbm.at[idx(0)], rows0, sem0)

        def body(p, carry):
            i = 2 * p
            pltpu.async_copy(hidden_hbm.at[idx(i + 1)], rows1, sem1)
            pltpu.make_async_copy(hidden_hbm.at[idx(i)], rows0, sem0).wait()
            pltpu.sync_copy(rows0, out_hbm.at[pl.ds(base + i * GCHUNK, GCHUNK)])

            @pl.when(p < nchunks // 2 - 1)
            def _():
                pltpu.async_copy(hidden_hbm.at[idx(i + 2)], rows0, sem0)

            pltpu.make_async_copy(hidden_hbm.at[idx(i + 1)], rows1, sem1).wait()
            pltpu.sync_copy(rows1,
                            out_hbm.at[pl.ds(base + (i + 1) * GCHUNK, GCHUNK)])
            return carry

        lax.fori_loop(0, nchunks // 2, body, 0)

    return _sc_gather_rows


@functools.lru_cache(maxsize=None)
def _sc_combine_kernel():
    mesh = plsc.VectorSubcoreMesh(core_axis_name="c", subcore_axis_name="s")

    @functools.partial(
        pl.kernel,
        mesh=mesh,
        out_type=jax.ShapeDtypeStruct((T, D), jnp.float32),
        scratch_types=[
            pltpu.VMEM((T // NW * TOPK,), jnp.int32),
            pltpu.VMEM((TCHUNK * TOPK, D), jnp.float32),
            pltpu.VMEM((TCHUNK * TOPK, D), jnp.float32),
            pltpu.VMEM((TCHUNK, D), jnp.float32),
            pltpu.SemaphoreType.DMA,
            pltpu.SemaphoreType.DMA,
        ],
    )
    def _sc_combine(down_hbm, cidx_hbm, out_hbm,
                    idx_all, rows0, rows1, acc_v, sem0, sem1):
        wid = lax.axis_index("s") * 2 + lax.axis_index("c")
        tok_per_w = T // NW
        nchunks = tok_per_w // TCHUNK
        base = wid * tok_per_w

        pltpu.sync_copy(cidx_hbm.at[pl.ds(base * TOPK, tok_per_w * TOPK)],
                        idx_all)

        def idx(i):
            return idx_all.at[pl.ds(i * TCHUNK * TOPK, TCHUNK * TOPK)]

        def reduce_store(rows_v, tok0):
            def lanes(j, c2):
                col = j * 16
                for t in range(TCHUNK):
                    s = rows_v[t * TOPK, pl.ds(col, 16)]
                    for k in range(1, TOPK):
                        s = s + rows_v[t * TOPK + k, pl.ds(col, 16)]
                    acc_v[t, pl.ds(col, 16)] = s
                return c2

            lax.fori_loop(0, D // 16, lanes, 0)
            pltpu.sync_copy(acc_v, out_hbm.at[pl.ds(tok0, TCHUNK)])

        pltpu.async_copy(down_hbm.at[idx(0)], rows0, sem0)

        def body(p, carry):
            i = 2 * p
            pltpu.async_copy(down_hbm.at[idx(i + 1)], rows1, sem1)
            pltpu.make_async_copy(down_hbm.at[idx(i)], rows0, sem0).wait()
            reduce_store(rows0, base + i * TCHUNK)

            @pl.when(p < nchunks // 2 - 1)
            def _():
                pltpu.async_copy(down_hbm.at[idx(i + 2)], rows0, sem0)

            pltpu.make_async_copy(down_hbm.at[idx(i + 1)], rows1, sem1).wait()
            reduce_store(rows1, base + (i + 1) * TCHUNK)
            return carry

        lax.fori_loop(0, nchunks // 2, body, 0)

    return _sc_combine


def _ffn_body(e_ref, x_ref, w_ref, wg_ref, wu_ref, wd_ref, o_ref,
              xb_s, g_s, h_s):
    i = pl.program_id(0)
    j = pl.program_id(1)
    live = i < e_ref[NT]

    @pl.when(live & (j == 0))
    def _():
        xb_s[...] = x_ref[...].astype(jnp.bfloat16)
        g_s[...] = jnp.dot(xb_s[...], wg_ref[0].astype(jnp.bfloat16),
                           preferred_element_type=jnp.float32)

    @pl.when(live & (j == 1))
    def _():
        g = g_s[...]
        u = jnp.dot(xb_s[...], wu_ref[0].astype(jnp.bfloat16),
                    preferred_element_type=jnp.float32)
        h_s[...] = (g * jax.nn.sigmoid(g) * u).astype(jnp.bfloat16)

    @pl.when(live & (j == 2))
    def _():
        d = jnp.dot(h_s[...], wd_ref[0].astype(jnp.bfloat16),
                    preferred_element_type=jnp.float32)
        o_ref[...] = d * w_ref[0, 0, :][:, None]


def _tc_ffn(tile_meta, x_padded, wvec3, Wg, Wu, Wd):
    def live(i, e):
        return jnp.minimum(i, e[NT] - 1)

    def ahead(i, j, e, thr):
        nxt = jnp.minimum(i + (j >= thr).astype(jnp.int32), NT - 1)
        return e[nxt]

    grid_spec = pltpu.PrefetchScalarGridSpec(
        num_scalar_prefetch=1,
        grid=(NT, 3),
        in_specs=[
            pl.BlockSpec((TM, D), lambda i, j, e: (live(i, e), 0)),
            pl.BlockSpec((1, 1, TM), lambda i, j, e: (live(i, e), 0, 0)),
            pl.BlockSpec((1, D, F), lambda i, j, e: (ahead(i, j, e, 1), 0, 0)),
            pl.BlockSpec((1, D, F), lambda i, j, e: (ahead(i, j, e, 2), 0, 0)),
            pl.BlockSpec((1, F, D), lambda i, j, e: (e[i], 0, 0)),
        ],
        out_specs=pl.BlockSpec((TM, D), lambda i, j, e: (live(i, e), 0)),
        scratch_shapes=[
            pltpu.VMEM((TM, D), jnp.bfloat16),
            pltpu.VMEM((TM, F), jnp.float32),
            pltpu.VMEM((TM, F), jnp.bfloat16),
        ],
    )
    return pl.pallas_call(
        _ffn_body,
        grid_spec=grid_spec,
        out_shape=jax.ShapeDtypeStruct((NPAD, D), jnp.float32),
        compiler_params=pltpu.CompilerParams(
            dimension_semantics=("arbitrary", "arbitrary"),
            vmem_limit_bytes=100 * 1024 * 1024,
        ),
    )(tile_meta, x_padded, wvec3, Wg, Wu, Wd)


def kernel(hidden_states, top_k_index, top_k_weights, Wg, Wu, Wd):
    tok_idx, wvec, cidx, tile_meta = _routing(top_k_index, top_k_weights)
    x_padded = _sc_gather_kernel()(hidden_states, tok_idx)
    down = _tc_ffn(tile_meta, x_padded, wvec.reshape(NT, 1, TM), Wg, Wu, Wd)
    return _sc_combine_kernel()(down, cidx)


# speedup vs baseline: 1.2415x; 1.2415x over previous
import functools

import jax
import jax.numpy as jnp
from jax import lax
from jax.experimental import pallas as pl
from jax.experimental.pallas import tpu as pltpu
from jax.experimental.pallas import tpu_sc as plsc

E = 64
TOPK = 8
D = 2048
F = 1024
T = 4096
N = T * TOPK
TM = 256
NT = N // TM + E
NPAD = NT * TM
NW = 32
TOKCH = 16
NCH = (T // NW) // TOKCH
TCHUNK = 2


def _routing(top_k_index, top_k_weights):
    flat_e = top_k_index.reshape(-1).astype(jnp.int32)
    BB, LL = 256, N // 256
    e2 = flat_e.reshape(BB, LL).T
    onehot = (e2[:, :, None] ==
              jnp.arange(E, dtype=jnp.int32)[None, None, :]
              ).astype(jnp.bfloat16)
    tril_l = jnp.tril(jnp.ones((LL, LL), jnp.bfloat16), -1)
    cum_in_blk = jnp.dot(tril_l, onehot.reshape(LL, BB * E),
                         preferred_element_type=jnp.float32
                         ).reshape(LL, BB, E)
    bsum = jnp.sum(onehot.astype(jnp.float32), axis=0)
    tril_b = jnp.tril(jnp.ones((BB, BB), jnp.bfloat16), -1)
    bpre = jnp.dot(tril_b, bsum.astype(jnp.bfloat16),
                   preferred_element_type=jnp.float32)
    rank_full = cum_in_blk + bpre[None, :, :]
    rank = jnp.sum(rank_full * onehot.astype(jnp.float32),
                   axis=-1).T.reshape(N).astype(jnp.int32)
    counts = (bpre[BB - 1] + bsum[BB - 1]).astype(jnp.int32)
    tiles_per_e = (counts + TM - 1) // TM
    zero = jnp.zeros((1,), jnp.int32)
    pad_off = jnp.concatenate([zero, jnp.cumsum(tiles_per_e)[:-1] * TM])
    ppos = pad_off[flat_e] + rank
    cidx4 = (ppos.reshape(NW, T // NW, TOPK).transpose(0, 2, 1)
             .reshape(NW, TOPK, NCH, TOKCH))
    w4 = (top_k_weights.astype(jnp.float32)
          .reshape(NW, T // NW, TOPK).transpose(0, 2, 1)
          .reshape(NW, TOPK, NCH, TOKCH))
    cum_tiles = jnp.cumsum(tiles_per_e)
    total_tiles = cum_tiles[E - 1]
    te = jnp.searchsorted(cum_tiles, jnp.arange(NT, dtype=jnp.int32),
                          side="right").astype(jnp.int32)
    te = jnp.minimum(te, E - 1)
    last_e = te[jnp.maximum(total_tiles - 1, 0)]
    tile_expert = jnp.where(jnp.arange(NT, dtype=jnp.int32) < total_tiles,
                            te, last_e)
    tile_meta = jnp.concatenate([tile_expert, total_tiles[None]])
    return cidx4, w4, ppos, tile_meta


@functools.lru_cache(maxsize=None)
def _sc_dispatch_kernel():
    mesh = plsc.VectorSubcoreMesh(core_axis_name="c", subcore_axis_name="s")

    @functools.partial(
        pl.kernel,
        mesh=mesh,
        out_type=(jax.ShapeDtypeStruct((NPAD, D), jnp.float32),
                  jax.ShapeDtypeStruct((NPAD,), jnp.float32)),
        scratch_types=[
            pltpu.VMEM((TOPK, NCH, TOKCH), jnp.int32),
            pltpu.VMEM((TOPK, NCH, TOKCH), jnp.float32),
            pltpu.VMEM((TOKCH, D), jnp.float32),
            pltpu.VMEM((TOKCH, D), jnp.float32),
            pltpu.SemaphoreType.DMA,
            pltpu.SemaphoreType.DMA,
        ],
    )
    def _sc_dispatch(hidden_hbm, cidx4_hbm, w4_hbm, xout_hbm, wout_hbm,
                     idx_v, w_v, buf0, buf1, sem0, sem1):
        wid = lax.axis_index("s") * 2 + lax.axis_index("c")
        tok_base = wid * (T // NW)
        pltpu.sync_copy(cidx4_hbm.at[wid], idx_v)
        pltpu.sync_copy(w4_hbm.at[wid], w_v)

        def load(c, buf, sem):
            pltpu.async_copy(
                hidden_hbm.at[pl.ds(tok_base + c * TOKCH, TOKCH)], buf, sem)

        def wait_load(buf, sem):
            pltpu.make_async_copy(
                hidden_hbm.at[pl.ds(0, TOKCH)], buf, sem).wait()

        def scatter(c, buf, sem):
            for k in range(TOPK):
                pltpu.async_copy(buf, xout_hbm.at[idx_v.at[k, c]], sem)
                pltpu.async_copy(w_v.at[k, c], wout_hbm.at[idx_v.at[k, c]],
                                 sem)
            for k in range(TOPK):
                pltpu.make_async_copy(
                    buf, xout_hbm.at[idx_v.at[k, c]], sem).wait()
                pltpu.make_async_copy(
                    w_v.at[k, c], wout_hbm.at[idx_v.at[k, c]], sem).wait()

        load(0, buf0, sem0)

        def body(p, carry):
            i = 2 * p
            load(i + 1, buf1, sem1)
            wait_load(buf0, sem0)
            scatter(i, buf0, sem0)

            @pl.when(p < NCH // 2 - 1)
            def _():
                load(i + 2, buf0, sem0)

            wait_load(buf1, sem1)
            scatter(i + 1, buf1, sem1)
            return carry

        lax.fori_loop(0, NCH // 2, body, 0)

    return _sc_dispatch


@functools.lru_cache(maxsize=None)
def _sc_combine_kernel():
    mesh = plsc.VectorSubcoreMesh(core_axis_name="c", subcore_axis_name="s")

    @functools.partial(
        pl.kernel,
        mesh=mesh,
        out_type=jax.ShapeDtypeStruct((T, D), jnp.float32),
        scratch_types=[
            pltpu.VMEM((T // NW * TOPK,), jnp.int32),
            pltpu.VMEM((TCHUNK * TOPK, D), jnp.float32),
            pltpu.VMEM((TCHUNK * TOPK, D), jnp.float32),
            pltpu.VMEM((TCHUNK, D), jnp.float32),
            pltpu.SemaphoreType.DMA,
            pltpu.SemaphoreType.DMA,
        ],
    )
    def _sc_combine(down_hbm, cidx_hbm, out_hbm,
                    idx_all, rows0, rows1, acc_v, sem0, sem1):
        wid = lax.axis_index("s") * 2 + lax.axis_index("c")
        tok_per_w = T // NW
        nchunks = tok_per_w // TCHUNK
        base = wid * tok_per_w

        pltpu.sync_copy(cidx_hbm.at[pl.ds(base * TOPK, tok_per_w * TOPK)],
                        idx_all)

        def idx(i):
            return idx_all.at[pl.ds(i * TCHUNK * TOPK, TCHUNK * TOPK)]

        def reduce_store(rows_v, tok0):
            def lanes(j, c2):
                col = j * 16
                for t in range(TCHUNK):
                    s = rows_v[t * TOPK, pl.ds(col, 16)]
                    for k in range(1, TOPK):
                        s = s + rows_v[t * TOPK + k, pl.ds(col, 16)]
                    acc_v[t, pl.ds(col, 16)] = s
                return c2

            lax.fori_loop(0, D // 16, lanes, 0)
            pltpu.sync_copy(acc_v, out_hbm.at[pl.ds(tok0, TCHUNK)])

        pltpu.async_copy(down_hbm.at[idx(0)], rows0, sem0)

        def body(p, carry):
            i = 2 * p
            pltpu.async_copy(down_hbm.at[idx(i + 1)], rows1, sem1)
            pltpu.make_async_copy(down_hbm.at[idx(i)], rows0, sem0).wait()
            reduce_store(rows0, base + i * TCHUNK)

            @pl.when(p < nchunks // 2 - 1)
            def _():
                pltpu.async_copy(down_hbm.at[idx(i + 2)], rows0, sem0)

            pltpu.make_async_copy(down_hbm.at[idx(i + 1)], rows1, sem1).wait()
            reduce_store(rows1, base + (i + 1) * TCHUNK)
            return carry

        lax.fori_loop(0, nchunks // 2, body, 0)

    return _sc_combine


def _ffn_body(e_ref, x_ref, w_ref, wg_ref, wu_ref, wd_ref, o_ref):
    i = pl.program_id(0)

    @pl.when(i < e_ref[NT])
    def _():
        x = x_ref[...].astype(jnp.bfloat16)
        g = jnp.dot(x, wg_ref[0].astype(jnp.bfloat16),
                    preferred_element_type=jnp.float32)
        u = jnp.dot(x, wu_ref[0].astype(jnp.bfloat16),
                    preferred_element_type=jnp.float32)
        h = (g * jax.nn.sigmoid(g) * u).astype(jnp.bfloat16)
        d = jnp.dot(h, wd_ref[0].astype(jnp.bfloat16),
                    preferred_element_type=jnp.float32)
        o_ref[...] = d * w_ref[0, 0, :][:, None]


def _tc_ffn(tile_meta, x_padded, wvec3, Wg, Wu, Wd):
    def live(i, e):
        return jnp.minimum(i, e[NT] - 1)

    grid_spec = pltpu.PrefetchScalarGridSpec(
        num_scalar_prefetch=1,
        grid=(NT,),
        in_specs=[
            pl.BlockSpec((TM, D), lambda i, e: (live(i, e), 0)),
            pl.BlockSpec((1, 1, TM), lambda i, e: (live(i, e), 0, 0)),
            pl.BlockSpec((1, D, F), lambda i, e: (e[i], 0, 0)),
            pl.BlockSpec((1, D, F), lambda i, e: (e[i], 0, 0)),
            pl.BlockSpec((1, F, D), lambda i, e: (e[i], 0, 0)),
        ],
        out_specs=pl.BlockSpec((TM, D), lambda i, e: (live(i, e), 0)),
    )
    return pl.pallas_call(
        _ffn_body,
        grid_spec=grid_spec,
        out_shape=jax.ShapeDtypeStruct((NPAD, D), jnp.float32),
        compiler_params=pltpu.CompilerParams(
            dimension_semantics=("arbitrary",),
            vmem_limit_bytes=100 * 1024 * 1024,
        ),
    )(tile_meta, x_padded, wvec3, Wg, Wu, Wd)


def kernel(hidden_states, top_k_index, top_k_weights, Wg, Wu, Wd):
    cidx4, w4, cidx, tile_meta = _routing(top_k_index, top_k_weights)
    x_padded, wvec = _sc_dispatch_kernel()(hidden_states, cidx4, w4)
    down = _tc_ffn(tile_meta, x_padded, wvec.reshape(NT, 1, TM), Wg, Wu, Wd)
    return _sc_combine_kernel()(down, cidx)
